# Initial kernel scaffold; baseline (speedup 1.0000x reference)
import jax
import jax.numpy as jnp
from jax.experimental import pallas as pl


def kernel(Z, planes):
    raise NotImplementedError("write your pallas kernel here")


# speedup vs baseline: 3.1833x; 3.1833x over previous
import numpy as np
import jax
import jax.numpy as jnp
from jax.experimental import pallas as pl
from jax.experimental.pallas import tpu as pltpu

_BANDS = 16
_ROWS = 8
_SIM_THRESH = 0.5


def _prep_kernel(z_ref, planes_ref, w_ref, cbn_ref, cnb_ref, invc_ref, invr_ref):
    z = z_ref[...]
    planes = planes_ref[...]
    w = w_ref[...]

    s = jax.lax.dot_general(planes, z, (((1,), (1,)), ((), ())),
                            preferred_element_type=jnp.float32,
                            precision=jax.lax.Precision.HIGHEST)
    bits = (s >= 0.0).astype(jnp.float32)

    cbn_ref[...] = jax.lax.dot_general(
        w, bits, (((1,), (0,)), ((), ())),
        preferred_element_type=jnp.float32,
        precision=jax.lax.Precision.HIGHEST)
    cnb_ref[...] = jax.lax.dot_general(
        bits, w, (((0,), (1,)), ((), ())),
        preferred_element_type=jnp.float32,
        precision=jax.lax.Precision.HIGHEST)

    zsq = z * z
    ones_row = jnp.ones((1, z.shape[1]), dtype=jnp.float32)
    nsq_col = jax.lax.dot_general(zsq, ones_row, (((1,), (1,)), ((), ())),
                                  preferred_element_type=jnp.float32,
                                  precision=jax.lax.Precision.HIGHEST)
    nsq_row = jax.lax.dot_general(ones_row, zsq, (((1,), (1,)), ((), ())),
                                  preferred_element_type=jnp.float32,
                                  precision=jax.lax.Precision.HIGHEST)
    invc_ref[...] = 1.0 / jnp.maximum(jnp.sqrt(nsq_col), 1e-8)
    invr_ref[...] = 1.0 / jnp.maximum(jnp.sqrt(nsq_row), 1e-8)


def _pairs_kernel(zi_ref, zj_ref, invi_ref, invj_ref, ci_ref, cj_ref, out_ref):
    gi = pl.program_id(0)
    gj = pl.program_id(1)
    tm = out_ref.shape[0]
    tn = out_ref.shape[1]

    dots = jax.lax.dot_general(zi_ref[...], zj_ref[...],
                               (((1,), (1,)), ((), ())),
                               preferred_element_type=jnp.float32)
    sim = dots * invi_ref[...] * invj_ref[...]

    ci = ci_ref[...]
    cj = cj_ref[...]
    m = ci[:, 0:1] == cj[0:1, :]
    for b in range(1, _BANDS):
        m = m | (ci[:, b:b + 1] == cj[b:b + 1, :])

    row = jax.lax.broadcasted_iota(jnp.int32, (tm, tn), 0) + gi * tm
    col = jax.lax.broadcasted_iota(jnp.int32, (tm, tn), 1) + gj * tn
    keep = m & (sim > _SIM_THRESH) & (row != col)
    out_ref[...] = jnp.where(keep, sim, 0.0)


def _pack_weights() -> np.ndarray:
    w = np.zeros((_BANDS, _BANDS * _ROWS), dtype=np.float32)
    for b in range(_BANDS):
        for r in range(_ROWS):
            w[b, b * _ROWS + r] = float(1 << r)
    return w


def kernel(Z, planes):
    N, D = Z.shape
    NC = 512
    TM = TN = 512
    w = jnp.asarray(_pack_weights())

    cbn, cnb, invc, invr = pl.pallas_call(
        _prep_kernel,
        grid=(N // NC,),
        in_specs=[
            pl.BlockSpec((NC, D), lambda i: (i, 0)),
            pl.BlockSpec((_BANDS * _ROWS, D), lambda i: (0, 0)),
            pl.BlockSpec((_BANDS, _BANDS * _ROWS), lambda i: (0, 0)),
        ],
        out_specs=[
            pl.BlockSpec((_BANDS, NC), lambda i: (0, i)),
            pl.BlockSpec((NC, _BANDS), lambda i: (i, 0)),
            pl.BlockSpec((NC, 1), lambda i: (i, 0)),
            pl.BlockSpec((1, NC), lambda i: (0, i)),
        ],
        out_shape=[
            jax.ShapeDtypeStruct((_BANDS, N), jnp.float32),
            jax.ShapeDtypeStruct((N, _BANDS), jnp.float32),
            jax.ShapeDtypeStruct((N, 1), jnp.float32),
            jax.ShapeDtypeStruct((1, N), jnp.float32),
        ],
    )(Z, planes, w)

    out = pl.pallas_call(
        _pairs_kernel,
        grid=(N // TM, N // TN),
        in_specs=[
            pl.BlockSpec((TM, D), lambda i, j: (i, 0)),
            pl.BlockSpec((TN, D), lambda i, j: (j, 0)),
            pl.BlockSpec((TM, 1), lambda i, j: (i, 0)),
            pl.BlockSpec((1, TN), lambda i, j: (0, j)),
            pl.BlockSpec((TM, _BANDS), lambda i, j: (i, 0)),
            pl.BlockSpec((_BANDS, TN), lambda i, j: (0, j)),
        ],
        out_specs=pl.BlockSpec((TM, TN), lambda i, j: (i, j)),
        out_shape=jax.ShapeDtypeStruct((N, N), jnp.float32),
        compiler_params=pltpu.CompilerParams(
            dimension_semantics=("parallel", "parallel"),
        ),
    )(Z, Z, invc, invr, cnb, cbn)
    return out
